# baseline (speedup 1.0000x reference)
import functools

import jax
import jax.numpy as jnp
from jax import lax
from jax.experimental import pallas as pl
from jax.experimental.pallas import tpu as pltpu
from jax.experimental.pallas import tpu_sc as plsc

N_ETHNICITIES = 1000
EMBED_DIM = 32
BATCH = 16384

_info = plsc.get_sparse_core_info()
_NC, _NS, _L = _info.num_cores, _info.num_subcores, _info.num_lanes
_NW = _NC * _NS
_B_PER_W = BATCH // _NW
_N_CHUNKS = 4
_CHUNK = _B_PER_W // _N_CHUNKS
_G_PER_CHUNK = _CHUNK // _L


@functools.partial(
    pl.kernel,
    mesh=plsc.VectorSubcoreMesh(core_axis_name="c", subcore_axis_name="s"),
    out_type=jax.ShapeDtypeStruct((EMBED_DIM, BATCH), jnp.float32),
    scratch_types=[
        pltpu.VMEM((_N_CHUNKS, _CHUNK), jnp.int32),
        pltpu.VMEM((_B_PER_W, EMBED_DIM), jnp.float32),
        pltpu.VMEM((EMBED_DIM, _B_PER_W), jnp.float32),
        pltpu.SemaphoreType.DMA,
        pltpu.SemaphoreType.DMA,
        pltpu.SemaphoreType.DMA,
        pltpu.SemaphoreType.DMA,
        pltpu.SemaphoreType.DMA,
    ],
    compiler_params=pltpu.CompilerParams(
        use_tc_tiling_on_sc=False,
        needs_layout_passes=False,
    ),
)
def _lookup_kernel(
    idx_hbm, table_hbm, out_hbm, idx_v, rows_v, trows_v, s0, s1, s2, s3, so
):
    wid = lax.axis_index("s") * _NC + lax.axis_index("c")
    base = wid * _B_PER_W
    sems = [s0, s1, s2, s3]
    lanes = lax.iota(jnp.int32, _L)

    def start_gather(c):
        pltpu.sync_copy(
            idx_hbm.at[pl.ds(base + c * _CHUNK, _CHUNK)], idx_v.at[c]
        )
        return pltpu.async_copy(
            table_hbm.at[idx_v.at[c]],
            rows_v.at[pl.ds(c * _CHUNK, _CHUNK)],
            sems[c],
        )

    gathers = [start_gather(0)]
    out_copies = []
    for c in range(_N_CHUNKS):
        if c + 1 < _N_CHUNKS:
            gathers.append(start_gather(c + 1))
        gathers[c].wait()

        def body(g, _):
            bvec = c * _CHUNK + g * _L + lanes

            def dbody(d8, _):
                for u in range(8):
                    dvec = jnp.bitwise_and(d8 * 8 + u + lanes, EMBED_DIM - 1)
                    vals = plsc.load_gather(rows_v, [bvec, dvec])
                    plsc.store_scatter(trows_v, [dvec, bvec], vals)
                return ()

            lax.fori_loop(0, EMBED_DIM // 8, dbody, ())
            return ()

        lax.fori_loop(0, _G_PER_CHUNK, body, ())
        out_copies.append(
            pltpu.async_copy(
                trows_v.at[:, pl.ds(c * _CHUNK, _CHUNK)],
                out_hbm.at[:, pl.ds(base + c * _CHUNK, _CHUNK)],
                so,
            )
        )
    for cp in out_copies:
        cp.wait()


def kernel(ethnicity_idx, embedding_table):
    out_t = _lookup_kernel(ethnicity_idx.astype(jnp.int32), embedding_table)
    return out_t.T


# speedup vs baseline: 1.1163x; 1.1163x over previous
import functools

import jax
import jax.numpy as jnp
from jax import lax
from jax.experimental import pallas as pl
from jax.experimental.pallas import tpu as pltpu
from jax.experimental.pallas import tpu_sc as plsc

N_ETHNICITIES = 1000
EMBED_DIM = 32
BATCH = 16384

_info = plsc.get_sparse_core_info()
_NC, _NS, _L = _info.num_cores, _info.num_subcores, _info.num_lanes
_NW = _NC * _NS
_B_PER_W = BATCH // _NW
_N_GROUPS = _B_PER_W // _L
_TR = EMBED_DIM // 8
_TC = BATCH // 128
_TC_PER_W = _B_PER_W // 128


@functools.partial(
    pl.kernel,
    mesh=plsc.VectorSubcoreMesh(core_axis_name="c", subcore_axis_name="s"),
    out_type=jax.ShapeDtypeStruct((_TR, _TC, 8, 128), jnp.float32),
    scratch_types=[
        pltpu.VMEM((_B_PER_W,), jnp.int32),
        pltpu.VMEM((_B_PER_W, EMBED_DIM), jnp.float32),
        pltpu.VMEM((EMBED_DIM, _B_PER_W), jnp.float32),
        pltpu.SemaphoreType.DMA,
        pltpu.SemaphoreType.DMA,
    ],
    compiler_params=pltpu.CompilerParams(
        use_tc_tiling_on_sc=False,
        needs_layout_passes=False,
        skip_device_barrier=True,
    ),
)
def _lookup_kernel(idx_hbm, table_hbm, out_hbm, idx_v, rows_v, trows_v, sem, so):
    wid = lax.axis_index("s") * _NC + lax.axis_index("c")
    base = wid * _B_PER_W
    pltpu.sync_copy(idx_hbm.at[pl.ds(base, _B_PER_W)], idx_v)
    pltpu.async_copy(table_hbm.at[idx_v], rows_v, sem).wait()

    lanes = lax.iota(jnp.int32, _L)

    def body(g, _):
        bvec = g * _L + lanes

        def dbody(d8, _):
            for u in range(8):
                dvec = jnp.bitwise_and(d8 * 8 + u + lanes, EMBED_DIM - 1)
                vals = plsc.load_gather(rows_v, [bvec, dvec])
                plsc.store_scatter(trows_v, [dvec, bvec], vals)
            return ()

        lax.fori_loop(0, EMBED_DIM // 8, dbody, ())
        return ()

    lax.fori_loop(0, _N_GROUPS, body, ())

    copies = []
    for i in range(_TR):
        for jl in range(_TC_PER_W):
            copies.append(
                pltpu.async_copy(
                    trows_v.at[pl.ds(8 * i, 8), pl.ds(jl * 128, 128)],
                    out_hbm.at[i, wid * _TC_PER_W + jl],
                    so,
                )
            )
    for cp in copies:
        cp.wait()


def kernel(ethnicity_idx, embedding_table):
    chunks = _lookup_kernel(ethnicity_idx.astype(jnp.int32), embedding_table)
    return chunks.transpose(1, 3, 0, 2).reshape(BATCH, EMBED_DIM)
